# baseline (speedup 1.0000x reference)
import jax
import jax.numpy as jnp
from jax.experimental import pallas as pl
from jax.experimental.pallas import tpu as pltpu

_IN_FEATURES = 10
_OUT_FEATURES = 5


def _linear_body(x_ref, wT_ref, b_ref, o_ref):
    y = jnp.dot(x_ref[...], wT_ref[...], preferred_element_type=jnp.float32)
    o_ref[...] = (y + b_ref[...]).astype(o_ref.dtype)


def kernel(x, weight, bias):
    B = x.shape[0]
    wT = weight.T
    b2 = bias.reshape(1, _OUT_FEATURES)

    TB = min(8192, B)
    grid = (pl.cdiv(B, TB),)

    return pl.pallas_call(
        _linear_body,
        out_shape=jax.ShapeDtypeStruct((B, _OUT_FEATURES), x.dtype),
        grid=grid,
        in_specs=[
            pl.BlockSpec((TB, _IN_FEATURES), lambda i: (i, 0)),
            pl.BlockSpec((_IN_FEATURES, _OUT_FEATURES), lambda i: (0, 0)),
            pl.BlockSpec((1, _OUT_FEATURES), lambda i: (0, 0)),
        ],
        out_specs=pl.BlockSpec((TB, _OUT_FEATURES), lambda i: (i, 0)),
        compiler_params=pltpu.CompilerParams(
            dimension_semantics=("parallel",),
            vmem_limit_bytes=64 * 1024 * 1024,
        ),
    )(x, wT, b2)
